# baseline (speedup 1.0000x reference)
import functools

import jax
import jax.numpy as jnp
from jax import lax
from jax.experimental import pallas as pl
from jax.experimental.pallas import tpu as pltpu
from jax.experimental.pallas import tpu_sc as plsc

_CHUNK = 128
_NBUF = 6
_DP = 128
_TG = 16


def _sc_gather_art(art512, art_tile_idx, batch):
    per_w = batch // 32
    chunk = 64
    nbuf = 2
    n_chunk = per_w // chunk

    mesh = plsc.VectorSubcoreMesh(core_axis_name="c", subcore_axis_name="s")

    @functools.partial(
        pl.kernel,
        out_type=jax.ShapeDtypeStruct((batch, 512), jnp.float32),
        mesh=mesh,
        scratch_types=[
            pltpu.VMEM((n_chunk, chunk), jnp.int32),
            pltpu.VMEM((nbuf, chunk, 512), jnp.float32),
            pltpu.SemaphoreType.DMA,
            pltpu.SemaphoreType.DMA,
        ],
    )
    def sc_kernel(art_hbm, aidx_hbm, art_out, aidx_v, rows_v, gsem, wsem):
        wid = lax.axis_index("s") * 2 + lax.axis_index("c")
        base = wid * per_w
        pltpu.sync_copy(aidx_hbm.at[wid], aidx_v)

        @pl.loop(0, n_chunk, step=nbuf)
        def _(g0):
            gathers = [
                pltpu.async_copy(art_hbm.at[aidx_v.at[g0 + b]], rows_v.at[b],
                                 gsem)
                for b in range(nbuf)
            ]
            for cp in gathers:
                cp.wait()
            writes = [
                pltpu.async_copy(
                    rows_v.at[b],
                    art_out.at[pl.ds(base + (g0 + b) * chunk, chunk)], wsem)
                for b in range(nbuf)
            ]
            for cp in writes:
                cp.wait()

    return sc_kernel(art512, art_tile_idx)


def _sc_gather_lab(lab_p, lab_idx3d, n_lab):
    lab_rows_w = lab_idx3d.shape[1]

    mesh = plsc.VectorSubcoreMesh(core_axis_name="c", subcore_axis_name="s")

    @functools.partial(
        pl.kernel,
        out_type=jax.ShapeDtypeStruct((n_lab, _DP), jnp.float32),
        mesh=mesh,
        compiler_params=pltpu.CompilerParams(use_tc_tiling_on_sc=False),
        scratch_types=[
            pltpu.VMEM((lab_rows_w, _CHUNK), jnp.int32),
            pltpu.VMEM((_NBUF, _CHUNK, _DP), jnp.float32),
            pltpu.SemaphoreType.DMA,
            pltpu.SemaphoreType.DMA,
        ],
    )
    def sc_kernel(lab_hbm, lidx_hbm, lab_out, lidx_v, rows_v, gsem, wsem):
        wid = lax.axis_index("s") * 2 + lax.axis_index("c")
        pltpu.sync_copy(lidx_hbm.at[wid], lidx_v)
        lab_base = wid * lab_rows_w * _CHUNK

        @pl.loop(0, lab_rows_w, step=_NBUF)
        def _(c0):
            gathers = [
                pltpu.async_copy(lab_hbm.at[lidx_v.at[c0 + b]], rows_v.at[b],
                                 gsem)
                for b in range(_NBUF)
            ]
            for cp in gathers:
                cp.wait()
            writes = [
                pltpu.async_copy(
                    rows_v.at[b],
                    lab_out.at[pl.ds(lab_base + (c0 + b) * _CHUNK, _CHUNK)],
                    wsem)
                for b in range(_NBUF)
            ]
            for cp in writes:
                cp.wait()

    return sc_kernel(lab_p, lab_idx3d)


def _tc_select_art(art_tiles, hsel, batch, bbc):
    def body(a_ref, h_ref, out_ref):
        a = a_ref[...]
        h = h_ref[...]
        c_id = lax.broadcasted_iota(jnp.int32, (8, 512), 1) // 64
        s_id = lax.broadcasted_iota(jnp.int32, (8, 512), 0)
        expand = (c_id == s_id).astype(jnp.float32)
        lane = lax.broadcasted_iota(jnp.int32, (512, _DP), 0) % 64
        out_l = lax.broadcasted_iota(jnp.int32, (512, _DP), 1)
        compact = (lane == out_l).astype(jnp.float32)
        mask = jax.lax.dot_general(h, expand, (((1,), (0,)), ((), ())),
                                   preferred_element_type=jnp.float32)
        sel = jax.lax.dot_general(a * mask, compact,
                                  (((1,), (0,)), ((), ())),
                                  preferred_element_type=jnp.float32)
        out_ref[...] = sel

    return pl.pallas_call(
        body,
        grid=(batch // bbc,),
        in_specs=[
            pl.BlockSpec((bbc, 512), lambda i: (i, 0)),
            pl.BlockSpec((bbc, 8), lambda i: (i, 0)),
        ],
        out_specs=pl.BlockSpec((bbc, _DP), lambda i: (i, 0)),
        out_shape=jax.ShapeDtypeStruct((batch, _DP), jnp.float32),
    )(art_tiles, hsel)


def _tc_scores(art_rows, lab_rows, batch, k, bb):
    nb = batch // bb
    rows_o = bb // 128

    def body(a_ref, g_ref, dots_ref):
        a = a_ref[...]
        g = g_ref[...]
        p = a * g
        ones_row = jnp.ones((1, _DP), dtype=jnp.float32)
        s = jax.lax.dot_general(ones_row, p, (((1,), (1,)), ((), ())),
                                preferred_element_type=jnp.float32)
        dots_ref[...] = s.reshape(rows_o, 128)

    dots = pl.pallas_call(
        body,
        grid=(nb, k),
        in_specs=[
            pl.BlockSpec((bb, _DP), lambda i, j: (i, 0)),
            pl.BlockSpec((bb, _DP), lambda i, j: (j * nb + i, 0)),
        ],
        out_specs=pl.BlockSpec((rows_o, 128), lambda i, j: (j * nb + i, 0)),
        out_shape=jax.ShapeDtypeStruct((k * batch // 128, 128), jnp.float32),
    )(art_rows, lab_rows)

    def norm_body(a_ref, g_ref, norm_ref):
        @pl.when(pl.program_id(0) == 0)
        def _():
            norm_ref[...] = jnp.zeros_like(norm_ref)

        a = a_ref[...]
        g = g_ref[...]
        norm_ref[...] += jnp.reshape(jnp.sum(a * a) + jnp.sum(g * g), (1, 1))

    norm = pl.pallas_call(
        norm_body,
        grid=(nb,),
        in_specs=[
            pl.BlockSpec((bb, _DP), lambda i: (i, 0)),
            pl.BlockSpec((bb, _DP), lambda i: (i, 0)),
        ],
        out_specs=pl.BlockSpec((1, 1), lambda i: (0, 0)),
        out_shape=jax.ShapeDtypeStruct((1, 1), jnp.float32),
    )(art_rows, lab_rows)
    return dots, norm


def kernel(art_embed, lab_embed, artist_idx, label_idx, noise_idxs):
    batch = artist_idx.shape[0]
    d = art_embed.shape[1]
    n_neg = noise_idxs.shape[1]
    k = n_neg + 1

    art512 = art_embed.reshape(art_embed.shape[0] // 8, 8 * d)
    aidx = artist_idx.astype(jnp.int32)
    art_tile_idx = (aidx // 8).reshape(32, -1, 64)
    hsel = jax.nn.one_hot(aidx % 8, 8, dtype=jnp.float32)

    lab_p = jnp.pad(lab_embed, ((0, 0), (0, _DP - d)))
    lab_all = jnp.concatenate(
        [label_idx[None, :], noise_idxs.T.astype(jnp.int32)], axis=0)
    lab_idx3d = lab_all.reshape(32, -1, _CHUNK)

    art_tiles = _sc_gather_art(art512, art_tile_idx, batch)
    lab_rows = _sc_gather_lab(lab_p, lab_idx3d, k * batch)

    art_c = _tc_select_art(art_tiles, hsel, batch, bbc=2048)
    dots, norm = _tc_scores(art_c, lab_rows, batch, k, bb=2048)

    dots2 = dots.reshape(k, batch)
    scores = dots2[0][:, None]
    noise_scores = dots2[1:].T
    embed_norm = norm[0, 0] / jnp.float32(batch * d)
    return scores, noise_scores, embed_norm
